# Initial kernel scaffold; baseline (speedup 1.0000x reference)
import jax
import jax.numpy as jnp
from jax.experimental import pallas as pl


def kernel(x, edge_attr, edge_index, params):
    raise NotImplementedError("write your pallas kernel here")


# speedup vs baseline: 6.4028x; 6.4028x over previous
import functools

import jax
import jax.numpy as jnp
from jax import lax
from jax.experimental import pallas as pl
from jax.experimental.pallas import tpu as pltpu
from jax.experimental.pallas import tpu_sc as plsc

N_NODES = 100000
N_PAD = 102400
E = 3200000
F = 16
EROWS = E * F // 128
NC, NS = 2, 16
NW = NC * NS
EPW = E // NW
BLK = 1000
SUB = 125
NSUB = BLK // SUB
NBLK = EPW // BLK
TPC = N_PAD // NS
EBR = 1000
NBR = 1024

_MM = functools.partial(lax.dot, precision=lax.Precision.HIGHEST,
                        preferred_element_type=jnp.float32)


def _padr(w, n):
    return jnp.pad(w, ((0, n - w.shape[0]), (0, 0)))


def _padc(w, n):
    return jnp.pad(w, ((0, 0), (0, n - w.shape[1])))


def _k8(w16):
    return jnp.kron(jnp.eye(8, dtype=jnp.float32), w16.astype(jnp.float32))


def _b8(b16):
    return jnp.tile(b16.astype(jnp.float32), (8,))[None, :]


def _sc_mesh():
    return plsc.VectorSubcoreMesh(core_axis_name="c", subcore_axis_name="s",
                                  num_cores=NC, num_subcores=NS)


_SC_PARAMS = pltpu.CompilerParams(use_tc_tiling_on_sc=False)


def _gather_body(tbl, idx, out, idx_v, rows_v, sem):
    c = lax.axis_index("c")
    s = lax.axis_index("s")
    wid = s * NC + c

    def body(b, carry):
        blk = wid * NBLK + b
        pltpu.sync_copy(idx.at[pl.ds(blk * NSUB, NSUB)], idx_v)
        descs = []
        for j in range(NSUB):
            descs.append(pltpu.async_copy(
                tbl.at[idx_v.at[j]],
                rows_v.at[pl.ds(j * SUB, SUB)], sem))
        for d in descs:
            d.wait()
        pltpu.sync_copy(rows_v, out.at[pl.ds(blk * BLK, BLK)])
        return carry

    lax.fori_loop(0, NBLK, body, 0)


def _sc_gather(table, idx2d):
    k = pl.kernel(
        _gather_body,
        out_type=jax.ShapeDtypeStruct((E, F), jnp.float32),
        mesh=_sc_mesh(),
        scratch_types=[
            pltpu.VMEM((NSUB, SUB), jnp.int32),
            pltpu.VMEM((BLK, F), jnp.float32),
            pltpu.SemaphoreType.DMA,
        ],
        compiler_params=_SC_PARAMS,
    )
    return k(table, idx2d)


def _scatter_body(vals, idx, zro, out, idx_v, vals_v, acc):
    c = lax.axis_index("c")
    s = lax.axis_index("s")
    wid = s * NC + c
    pltpu.sync_copy(zro.at[pl.ds(s * TPC, TPC)], acc.at[pl.ds(s * TPC, TPC)])
    plsc.subcore_barrier()

    def body(b, carry):
        blk = wid * NBLK + b
        pltpu.sync_copy(idx.at[pl.ds(blk * NSUB, NSUB)], idx_v)
        pltpu.sync_copy(vals.at[pl.ds(blk * BLK, BLK)], vals_v)
        for j in range(NSUB):
            pltpu.sync_copy(vals_v.at[pl.ds(j * SUB, SUB)],
                            acc.at[idx_v.at[j]], add=True)
        return carry

    lax.fori_loop(0, NBLK, body, 0)
    plsc.subcore_barrier()
    pltpu.sync_copy(acc.at[pl.ds(s * TPC, TPC)],
                    out.at[c, pl.ds(s * TPC, TPC)])


def _sc_scatter_add(vals16, idx2d, zeros_hbm):
    k = pl.kernel(
        _scatter_body,
        out_type=jax.ShapeDtypeStruct((NC, N_PAD, F), jnp.float32),
        mesh=_sc_mesh(),
        scratch_types=[
            pltpu.VMEM((NSUB, SUB), jnp.int32),
            pltpu.VMEM((BLK, F), jnp.float32),
            pltpu.VMEM_SHARED((N_PAD, F), jnp.float32),
        ],
        compiler_params=_SC_PARAMS,
    )
    return k(vals16, idx2d, zeros_hbm)


def _embed_body(x_ref, w_ref, b_ref, o_ref):
    o_ref[...] = _MM(x_ref[...], w_ref[...]) + b_ref[...]


def _tc_embed(x_pad, wnp, bnp):
    return pl.pallas_call(
        _embed_body,
        grid=(N_PAD // NBR,),
        in_specs=[
            pl.BlockSpec((NBR, 12), lambda i: (i, 0)),
            pl.BlockSpec((12, F), lambda i: (0, 0)),
            pl.BlockSpec((1, F), lambda i: (0, 0)),
        ],
        out_specs=pl.BlockSpec((NBR, F), lambda i: (i, 0)),
        out_shape=jax.ShapeDtypeStruct((N_PAD, F), jnp.float32),
    )(x_pad, wnp, bnp)


def _edge2_body(ein, gs, gd, wa, wb, wc, w2, na, nb, b1, b2, er_ref, u_ref):
    h1 = jnp.maximum(
        _MM(gs[...], wa[...]) + _MM(ein[...], wb[...])
        + _MM(gd[...], wc[...]) + b1[...], 0.0)
    er = jnp.maximum(_MM(h1, w2[...]) + b2[...], 0.0)
    er_ref[...] = er
    u_ref[...] = _MM(gs[...], na[...]) + _MM(er, nb[...])


def _edge3_body(ein, gs, gd, wa, wb, wc, w2, w3, na, nb, b1, b2, b3,
                er_ref, u_ref):
    h1 = jnp.maximum(
        _MM(gs[...], wa[...]) + _MM(ein[...], wb[...])
        + _MM(gd[...], wc[...]) + b1[...], 0.0)
    h2 = jnp.maximum(_MM(h1, w2[...]) + b2[...], 0.0)
    er = _MM(h2, w3[...]) + b3[...]
    er_ref[...] = er
    u_ref[...] = _MM(gs[...], na[...]) + _MM(er, nb[...])


def _tc_edge(body, ein_p, gs_p, gd_p, mats, vecs):
    nmat, nvec = len(mats), len(vecs)
    big = pl.BlockSpec((EBR, 128), lambda i: (i, 0))
    wsp = pl.BlockSpec((128, 128), lambda i: (0, 0))
    bsp = pl.BlockSpec((1, 128), lambda i: (0, 0))
    return pl.pallas_call(
        body,
        grid=(EROWS // EBR,),
        in_specs=[big, big, big] + [wsp] * nmat + [bsp] * nvec,
        out_specs=[big, big],
        out_shape=[jax.ShapeDtypeStruct((EROWS, 128), jnp.float32)] * 2,
    )(ein_p, gs_p, gd_p, *mats, *vecs)


def _node2_body(acc, n2, c1, c2, o_ref):
    s = acc[0] + acc[1]
    h = jnp.maximum(s + c1[...], 0.0)
    o_ref[...] = jnp.maximum(_MM(h, n2[...]) + c2[...], 0.0)


def _node3_body(acc, n2, n3, c1, c2, c3, o_ref):
    s = acc[0] + acc[1]
    h1 = jnp.maximum(s + c1[...], 0.0)
    h2 = jnp.maximum(_MM(h1, n2[...]) + c2[...], 0.0)
    o_ref[...] = _MM(h2, n3[...]) + c3[...]


def _tc_node(body, accs, mats, vecs):
    asp = pl.BlockSpec((NC, NBR, F), lambda i: (0, i, 0))
    wsp = pl.BlockSpec((F, F), lambda i: (0, 0))
    bsp = pl.BlockSpec((1, F), lambda i: (0, 0))
    return pl.pallas_call(
        body,
        grid=(N_PAD // NBR,),
        in_specs=[asp] + [wsp] * len(mats) + [bsp] * len(vecs),
        out_specs=pl.BlockSpec((NBR, F), lambda i: (i, 0)),
        out_shape=jax.ShapeDtypeStruct((N_PAD, F), jnp.float32),
    )(accs, *mats, *vecs)


def _prep(params):
    f32 = jnp.float32
    Wn, bn = params['emb_n']
    We, be = params['emb_e']
    (W1, b1), (W2, b2) = params['c1_e']
    (N1, c1), (N2, c2) = params['c1_n']
    (V1, d1), (V2, d2) = params['c2_e']
    (M1, e1), (M2, e2) = params['c2_n']
    (U1, f1), (U2, f2), (U3, f3) = params['c3_e']
    (P1, g1), (P2, g2), (P3, g3) = params['c3_n']

    p = {}
    p['emb_w'] = _padc(Wn.astype(f32), F)
    p['emb_b'] = _padc(bn.astype(f32)[None, :], F)

    p['e1_wa'] = _k8(_padr(W1[0:12], F))
    p['e1_wb'] = _k8(_padr(We @ W1[12:18], F))
    p['e1_wc'] = _k8(_padr(W1[18:30], F))
    p['e1_w2'] = _k8(W2)
    p['e1_b1'] = _b8(be @ W1[12:18] + b1)
    p['e1_b2'] = _b8(b2)
    p['e1_na'] = _k8(_padr(N1[0:12], F))
    p['e1_nb'] = _k8(N1[12:28])
    p['n1_w2'] = N2.astype(f32)
    p['n1_c1'] = c1.astype(f32)[None, :]
    p['n1_c2'] = c2.astype(f32)[None, :]

    p['e2_wa'] = _k8(V1[0:16])
    p['e2_wb'] = _k8(V1[16:32])
    p['e2_wc'] = _k8(V1[32:48])
    p['e2_w2'] = _k8(V2)
    p['e2_b1'] = _b8(d1)
    p['e2_b2'] = _b8(d2)
    p['e2_na'] = _k8(M1[0:16])
    p['e2_nb'] = _k8(M1[16:32])
    p['n2_w2'] = M2.astype(f32)
    p['n2_c1'] = e1.astype(f32)[None, :]
    p['n2_c2'] = e2.astype(f32)[None, :]

    p['e3_wa'] = _k8(U1[0:16])
    p['e3_wb'] = _k8(U1[16:32])
    p['e3_wc'] = _k8(U1[32:48])
    p['e3_w2'] = _k8(U2)
    p['e3_w3'] = _k8(_padc(U3, F))
    p['e3_b1'] = _b8(f1)
    p['e3_b2'] = _b8(f2)
    p['e3_b3'] = _b8(_padc(f3[None, :], F)[0])
    p['e3_na'] = _k8(P1[0:16])
    p['e3_nb'] = _k8(_padr(P1[16:19], F))
    p['n3_w2'] = P2.astype(f32)
    p['n3_w3'] = _padc(P3.astype(f32), F)
    p['n3_c1'] = g1.astype(f32)[None, :]
    p['n3_c2'] = g2.astype(f32)[None, :]
    p['n3_c3'] = _padc(g3.astype(f32)[None, :], F)
    return p


def _conv(tbl, ein_p, src2d, dst2d, zeros_hbm, p, tag, last):
    gs = _sc_gather(tbl, src2d).reshape(EROWS, 128)
    gd = _sc_gather(tbl, dst2d).reshape(EROWS, 128)
    if last:
        er_p, u_p = _tc_edge(
            _edge3_body, ein_p, gs, gd,
            [p['e3_wa'], p['e3_wb'], p['e3_wc'], p['e3_w2'], p['e3_w3'],
             p['e3_na'], p['e3_nb']],
            [p['e3_b1'], p['e3_b2'], p['e3_b3']])
    else:
        er_p, u_p = _tc_edge(
            _edge2_body, ein_p, gs, gd,
            [p[tag + '_wa'], p[tag + '_wb'], p[tag + '_wc'], p[tag + '_w2'],
             p[tag + '_na'], p[tag + '_nb']],
            [p[tag + '_b1'], p[tag + '_b2']])
    accs = _sc_scatter_add(u_p.reshape(E, F), dst2d, zeros_hbm)
    return er_p, accs


def kernel(x, edge_attr, edge_index, params):
    p = _prep(params)
    src2d = edge_index[0].astype(jnp.int32).reshape(E // SUB, SUB)
    dst2d = edge_index[1].astype(jnp.int32).reshape(E // SUB, SUB)
    zeros_hbm = jnp.zeros((N_PAD, F), jnp.float32)

    ea_p = jnp.pad(edge_attr.astype(jnp.float32),
                   ((0, 0), (0, F - 6))).reshape(EROWS, 128)
    x_pad = jnp.pad(x.astype(jnp.float32), ((0, N_PAD - N_NODES), (0, 0)))

    tbl1 = _tc_embed(x_pad, p['emb_w'], p['emb_b'])

    er1, acc1 = _conv(tbl1, ea_p, src2d, dst2d, zeros_hbm, p, 'e1', False)
    tbl2 = _tc_node(_node2_body, acc1, [p['n1_w2']],
                    [p['n1_c1'], p['n1_c2']])

    er2, acc2 = _conv(tbl2, er1, src2d, dst2d, zeros_hbm, p, 'e2', False)
    tbl3 = _tc_node(_node2_body, acc2, [p['n2_w2']],
                    [p['n2_c1'], p['n2_c2']])

    er3, acc3 = _conv(tbl3, er2, src2d, dst2d, zeros_hbm, p, 'e3', True)
    out_n = _tc_node(_node3_body, acc3, [p['n3_w2'], p['n3_w3']],
                     [p['n3_c1'], p['n3_c2'], p['n3_c3']])

    xn_out = out_n[:N_NODES, :3]
    xe_out = er3.reshape(E, F)[:, :3]
    return (xn_out, xe_out)


# speedup vs baseline: 9.5341x; 1.4891x over previous
import functools

import jax
import jax.numpy as jnp
from jax import lax
from jax.experimental import pallas as pl
from jax.experimental.pallas import tpu as pltpu
from jax.experimental.pallas import tpu_sc as plsc

N_NODES = 100000
N_PAD = 102400
E = 3200000
F = 16
EROWS = E * F // 128
NC, NS = 2, 16
NW = NC * NS
EPW = E // NW
BLK = 1000
SUB = 125
NSUB = BLK // SUB
NBLK = EPW // BLK
TPC = N_PAD // NS
EBR = 1000
NBR = 1024

_DOT = functools.partial(lax.dot, preferred_element_type=jnp.float32)


def _MM(x, w):
    xh = x.astype(jnp.bfloat16)
    xl = (x - xh.astype(jnp.float32)).astype(jnp.bfloat16)
    wh = w.astype(jnp.bfloat16)
    wl = (w - wh.astype(jnp.float32)).astype(jnp.bfloat16)
    return _DOT(xh, wh) + _DOT(xl, wh) + _DOT(xh, wl)


def _padr(w, n):
    return jnp.pad(w, ((0, n - w.shape[0]), (0, 0)))


def _padc(w, n):
    return jnp.pad(w, ((0, 0), (0, n - w.shape[1])))


def _k8(w16):
    return jnp.kron(jnp.eye(8, dtype=jnp.float32), w16.astype(jnp.float32))


def _b8(b16):
    return jnp.tile(b16.astype(jnp.float32), (8,))[None, :]


def _sc_mesh():
    return plsc.VectorSubcoreMesh(core_axis_name="c", subcore_axis_name="s",
                                  num_cores=NC, num_subcores=NS)


_SC_PARAMS = pltpu.CompilerParams(use_tc_tiling_on_sc=False)


def _gather_body(tbl, idx, out, idx_v, rows_v, sem):
    c = lax.axis_index("c")
    s = lax.axis_index("s")
    wid = s * NC + c

    def body(b, carry):
        blk = wid * NBLK + b
        pltpu.sync_copy(idx.at[pl.ds(blk * NSUB, NSUB)], idx_v)
        descs = []
        for j in range(NSUB):
            descs.append(pltpu.async_copy(
                tbl.at[idx_v.at[j]],
                rows_v.at[pl.ds(j * SUB, SUB)], sem))
        for d in descs:
            d.wait()
        pltpu.sync_copy(rows_v, out.at[pl.ds(blk * BLK, BLK)])
        return carry

    lax.fori_loop(0, NBLK, body, 0)


def _sc_gather(table, idx2d):
    k = pl.kernel(
        _gather_body,
        out_type=jax.ShapeDtypeStruct((E, F), jnp.float32),
        mesh=_sc_mesh(),
        scratch_types=[
            pltpu.VMEM((NSUB, SUB), jnp.int32),
            pltpu.VMEM((BLK, F), jnp.float32),
            pltpu.SemaphoreType.DMA,
        ],
        compiler_params=_SC_PARAMS,
    )
    return k(table, idx2d)


def _scatter_body(vals, idx, zro, out, idx_v, vals_v, acc):
    c = lax.axis_index("c")
    s = lax.axis_index("s")
    wid = s * NC + c
    pltpu.sync_copy(zro.at[pl.ds(s * TPC, TPC)], acc.at[pl.ds(s * TPC, TPC)])
    plsc.subcore_barrier()

    def body(b, carry):
        blk = wid * NBLK + b
        pltpu.sync_copy(idx.at[pl.ds(blk * NSUB, NSUB)], idx_v)
        pltpu.sync_copy(vals.at[pl.ds(blk * BLK, BLK)], vals_v)
        for j in range(NSUB):
            pltpu.sync_copy(vals_v.at[pl.ds(j * SUB, SUB)],
                            acc.at[idx_v.at[j]], add=True)
        return carry

    lax.fori_loop(0, NBLK, body, 0)
    plsc.subcore_barrier()
    pltpu.sync_copy(acc.at[pl.ds(s * TPC, TPC)],
                    out.at[c, pl.ds(s * TPC, TPC)])


def _sc_scatter_add(vals16, idx2d, zeros_hbm):
    k = pl.kernel(
        _scatter_body,
        out_type=jax.ShapeDtypeStruct((NC, N_PAD, F), jnp.float32),
        mesh=_sc_mesh(),
        scratch_types=[
            pltpu.VMEM((NSUB, SUB), jnp.int32),
            pltpu.VMEM((BLK, F), jnp.float32),
            pltpu.VMEM_SHARED((N_PAD, F), jnp.float32),
        ],
        compiler_params=_SC_PARAMS,
    )
    return k(vals16, idx2d, zeros_hbm)


def _embed_body(x_ref, w_ref, b_ref, o_ref):
    o_ref[...] = _MM(x_ref[...], w_ref[...]) + b_ref[...]


def _tc_embed(x_pad, wnp, bnp):
    return pl.pallas_call(
        _embed_body,
        grid=(N_PAD // NBR,),
        in_specs=[
            pl.BlockSpec((NBR, 12), lambda i: (i, 0)),
            pl.BlockSpec((12, F), lambda i: (0, 0)),
            pl.BlockSpec((1, F), lambda i: (0, 0)),
        ],
        out_specs=pl.BlockSpec((NBR, F), lambda i: (i, 0)),
        out_shape=jax.ShapeDtypeStruct((N_PAD, F), jnp.float32),
    )(x_pad, wnp, bnp)


def _edge2_body(ein, gs, gd, wa, wb, wc, w2, na, nb, b1, b2, er_ref, u_ref):
    h1 = jnp.maximum(
        _MM(gs[...], wa[...]) + _MM(ein[...], wb[...])
        + _MM(gd[...], wc[...]) + b1[...], 0.0)
    er = jnp.maximum(_MM(h1, w2[...]) + b2[...], 0.0)
    er_ref[...] = er
    u_ref[...] = _MM(gs[...], na[...]) + _MM(er, nb[...])


def _edge3_body(ein, gs, gd, wa, wb, wc, w2, w3, na, nb, b1, b2, b3,
                er_ref, u_ref):
    h1 = jnp.maximum(
        _MM(gs[...], wa[...]) + _MM(ein[...], wb[...])
        + _MM(gd[...], wc[...]) + b1[...], 0.0)
    h2 = jnp.maximum(_MM(h1, w2[...]) + b2[...], 0.0)
    er = _MM(h2, w3[...]) + b3[...]
    er_ref[...] = er
    u_ref[...] = _MM(gs[...], na[...]) + _MM(er, nb[...])


def _tc_edge(body, ein_p, gs_p, gd_p, mats, vecs):
    nmat, nvec = len(mats), len(vecs)
    big = pl.BlockSpec((EBR, 128), lambda i: (i, 0))
    wsp = pl.BlockSpec((128, 128), lambda i: (0, 0))
    bsp = pl.BlockSpec((1, 128), lambda i: (0, 0))
    return pl.pallas_call(
        body,
        grid=(EROWS // EBR,),
        in_specs=[big, big, big] + [wsp] * nmat + [bsp] * nvec,
        out_specs=[big, big],
        out_shape=[jax.ShapeDtypeStruct((EROWS, 128), jnp.float32)] * 2,
    )(ein_p, gs_p, gd_p, *mats, *vecs)


def _node2_body(acc, n2, c1, c2, o_ref):
    s = acc[0] + acc[1]
    h = jnp.maximum(s + c1[...], 0.0)
    o_ref[...] = jnp.maximum(_MM(h, n2[...]) + c2[...], 0.0)


def _node3_body(acc, n2, n3, c1, c2, c3, o_ref):
    s = acc[0] + acc[1]
    h1 = jnp.maximum(s + c1[...], 0.0)
    h2 = jnp.maximum(_MM(h1, n2[...]) + c2[...], 0.0)
    o_ref[...] = _MM(h2, n3[...]) + c3[...]


def _tc_node(body, accs, mats, vecs):
    asp = pl.BlockSpec((NC, NBR, F), lambda i: (0, i, 0))
    wsp = pl.BlockSpec((F, F), lambda i: (0, 0))
    bsp = pl.BlockSpec((1, F), lambda i: (0, 0))
    return pl.pallas_call(
        body,
        grid=(N_PAD // NBR,),
        in_specs=[asp] + [wsp] * len(mats) + [bsp] * len(vecs),
        out_specs=pl.BlockSpec((NBR, F), lambda i: (i, 0)),
        out_shape=jax.ShapeDtypeStruct((N_PAD, F), jnp.float32),
    )(accs, *mats, *vecs)


def _prep(params):
    f32 = jnp.float32
    Wn, bn = params['emb_n']
    We, be = params['emb_e']
    (W1, b1), (W2, b2) = params['c1_e']
    (N1, c1), (N2, c2) = params['c1_n']
    (V1, d1), (V2, d2) = params['c2_e']
    (M1, e1), (M2, e2) = params['c2_n']
    (U1, f1), (U2, f2), (U3, f3) = params['c3_e']
    (P1, g1), (P2, g2), (P3, g3) = params['c3_n']

    p = {}
    p['emb_w'] = _padc(Wn.astype(f32), F)
    p['emb_b'] = _padc(bn.astype(f32)[None, :], F)

    p['e1_wa'] = _k8(_padr(W1[0:12], F))
    p['e1_wb'] = _k8(_padr(We @ W1[12:18], F))
    p['e1_wc'] = _k8(_padr(W1[18:30], F))
    p['e1_w2'] = _k8(W2)
    p['e1_b1'] = _b8(be @ W1[12:18] + b1)
    p['e1_b2'] = _b8(b2)
    p['e1_na'] = _k8(_padr(N1[0:12], F))
    p['e1_nb'] = _k8(N1[12:28])
    p['n1_w2'] = N2.astype(f32)
    p['n1_c1'] = c1.astype(f32)[None, :]
    p['n1_c2'] = c2.astype(f32)[None, :]

    p['e2_wa'] = _k8(V1[0:16])
    p['e2_wb'] = _k8(V1[16:32])
    p['e2_wc'] = _k8(V1[32:48])
    p['e2_w2'] = _k8(V2)
    p['e2_b1'] = _b8(d1)
    p['e2_b2'] = _b8(d2)
    p['e2_na'] = _k8(M1[0:16])
    p['e2_nb'] = _k8(M1[16:32])
    p['n2_w2'] = M2.astype(f32)
    p['n2_c1'] = e1.astype(f32)[None, :]
    p['n2_c2'] = e2.astype(f32)[None, :]

    p['e3_wa'] = _k8(U1[0:16])
    p['e3_wb'] = _k8(U1[16:32])
    p['e3_wc'] = _k8(U1[32:48])
    p['e3_w2'] = _k8(U2)
    p['e3_w3'] = _k8(_padc(U3, F))
    p['e3_b1'] = _b8(f1)
    p['e3_b2'] = _b8(f2)
    p['e3_b3'] = _b8(_padc(f3[None, :], F)[0])
    p['e3_na'] = _k8(P1[0:16])
    p['e3_nb'] = _k8(_padr(P1[16:19], F))
    p['n3_w2'] = P2.astype(f32)
    p['n3_w3'] = _padc(P3.astype(f32), F)
    p['n3_c1'] = g1.astype(f32)[None, :]
    p['n3_c2'] = g2.astype(f32)[None, :]
    p['n3_c3'] = _padc(g3.astype(f32)[None, :], F)
    return p


def _conv(tbl, ein_p, src2d, dst2d, zeros_hbm, p, tag, last):
    gs = _sc_gather(tbl, src2d).reshape(EROWS, 128)
    gd = _sc_gather(tbl, dst2d).reshape(EROWS, 128)
    if last:
        er_p, u_p = _tc_edge(
            _edge3_body, ein_p, gs, gd,
            [p['e3_wa'], p['e3_wb'], p['e3_wc'], p['e3_w2'], p['e3_w3'],
             p['e3_na'], p['e3_nb']],
            [p['e3_b1'], p['e3_b2'], p['e3_b3']])
    else:
        er_p, u_p = _tc_edge(
            _edge2_body, ein_p, gs, gd,
            [p[tag + '_wa'], p[tag + '_wb'], p[tag + '_wc'], p[tag + '_w2'],
             p[tag + '_na'], p[tag + '_nb']],
            [p[tag + '_b1'], p[tag + '_b2']])
    accs = _sc_scatter_add(u_p.reshape(E, F), dst2d, zeros_hbm)
    return er_p, accs


def kernel(x, edge_attr, edge_index, params):
    p = _prep(params)
    src2d = edge_index[0].astype(jnp.int32).reshape(E // SUB, SUB)
    dst2d = edge_index[1].astype(jnp.int32).reshape(E // SUB, SUB)
    zeros_hbm = jnp.zeros((N_PAD, F), jnp.float32)

    ea_p = jnp.pad(edge_attr.astype(jnp.float32),
                   ((0, 0), (0, F - 6))).reshape(EROWS, 128)
    x_pad = jnp.pad(x.astype(jnp.float32), ((0, N_PAD - N_NODES), (0, 0)))

    tbl1 = _tc_embed(x_pad, p['emb_w'], p['emb_b'])

    er1, acc1 = _conv(tbl1, ea_p, src2d, dst2d, zeros_hbm, p, 'e1', False)
    tbl2 = _tc_node(_node2_body, acc1, [p['n1_w2']],
                    [p['n1_c1'], p['n1_c2']])

    er2, acc2 = _conv(tbl2, er1, src2d, dst2d, zeros_hbm, p, 'e2', False)
    tbl3 = _tc_node(_node2_body, acc2, [p['n2_w2']],
                    [p['n2_c1'], p['n2_c2']])

    er3, acc3 = _conv(tbl3, er2, src2d, dst2d, zeros_hbm, p, 'e3', True)
    out_n = _tc_node(_node3_body, acc3, [p['n3_w2'], p['n3_w3']],
                     [p['n3_c1'], p['n3_c2'], p['n3_c3']])

    xn_out = out_n[:N_NODES, :3]
    xe_out = er3.reshape(E, F)[:, :3]
    return (xn_out, xe_out)


# speedup vs baseline: 12.7381x; 1.3361x over previous
import functools

import jax
import jax.numpy as jnp
from jax import lax
from jax.experimental import pallas as pl
from jax.experimental.pallas import tpu as pltpu
from jax.experimental.pallas import tpu_sc as plsc

N_NODES = 100000
N_PAD = 102400
E = 3200000
F = 16
EROWS = E * F // 128
NC, NS = 2, 16
NW = NC * NS
EPW = E // NW
BLK = 1000
SUB = 125
NSUB = BLK // SUB
NBLK = EPW // BLK
TPC = N_PAD // NS
EBR = 1000
NBR = 1024

_DOT = functools.partial(lax.dot, preferred_element_type=jnp.float32)


def _MM(x, w):
    return _DOT(x.astype(jnp.bfloat16), w.astype(jnp.bfloat16))


def _padr(w, n):
    return jnp.pad(w, ((0, n - w.shape[0]), (0, 0)))


def _padc(w, n):
    return jnp.pad(w, ((0, 0), (0, n - w.shape[1])))


def _k8(w16):
    return jnp.kron(jnp.eye(8, dtype=jnp.float32), w16.astype(jnp.float32))


def _b8(b16):
    return jnp.tile(b16.astype(jnp.float32), (8,))[None, :]


def _sc_mesh():
    return plsc.VectorSubcoreMesh(core_axis_name="c", subcore_axis_name="s",
                                  num_cores=NC, num_subcores=NS)


_SC_PARAMS = pltpu.CompilerParams(use_tc_tiling_on_sc=False)
_SC_PARAMS_NL = pltpu.CompilerParams(use_tc_tiling_on_sc=False,
                                     needs_layout_passes=False)


def _gather_body(tbl, idx, out, idx_v, rows_v, sem):
    c = lax.axis_index("c")
    s = lax.axis_index("s")
    wid = s * NC + c

    def body(b, carry):
        blk = wid * NBLK + b
        pltpu.sync_copy(idx.at[pl.ds(blk * NSUB, NSUB)], idx_v)
        descs = []
        for j in range(NSUB):
            descs.append(pltpu.async_copy(
                tbl.at[idx_v.at[j]],
                rows_v.at[pl.ds(j * SUB, SUB)], sem))
        for d in descs:
            d.wait()
        pltpu.sync_copy(rows_v, out.at[pl.ds(blk * BLK, BLK)])
        return carry

    lax.fori_loop(0, NBLK, body, 0)


def _sc_gather(table, idx2d):
    k = pl.kernel(
        _gather_body,
        out_type=jax.ShapeDtypeStruct((E, F), jnp.float32),
        mesh=_sc_mesh(),
        scratch_types=[
            pltpu.VMEM((NSUB, SUB), jnp.int32),
            pltpu.VMEM((BLK, F), jnp.float32),
            pltpu.SemaphoreType.DMA,
        ],
        compiler_params=_SC_PARAMS,
    )
    return k(table, idx2d)


def _scatter_body(vals, idx, zro, out, idx_v, vals_v, acc):
    c = lax.axis_index("c")
    s = lax.axis_index("s")
    wid = s * NC + c
    pltpu.sync_copy(zro.at[pl.ds(s * TPC, TPC)], acc.at[pl.ds(s * TPC, TPC)])
    plsc.subcore_barrier()

    def body(b, carry):
        blk = wid * NBLK + b
        pltpu.sync_copy(idx.at[pl.ds(blk * NSUB, NSUB)], idx_v)
        pltpu.sync_copy(vals.at[pl.ds(blk * BLK, BLK)], vals_v)
        for j in range(NSUB):
            pltpu.sync_copy(vals_v.at[pl.ds(j * SUB, SUB)],
                            acc.at[idx_v.at[j]], add=True)
        return carry

    lax.fori_loop(0, NBLK, body, 0)
    plsc.subcore_barrier()
    pltpu.sync_copy(acc.at[pl.ds(s * TPC, TPC)],
                    out.at[c, pl.ds(s * TPC, TPC)])


def _sc_scatter_add(vals16, idx2d, zeros_hbm):
    k = pl.kernel(
        _scatter_body,
        out_type=jax.ShapeDtypeStruct((NC, N_PAD, F), jnp.float32),
        mesh=_sc_mesh(),
        scratch_types=[
            pltpu.VMEM((NSUB, SUB), jnp.int32),
            pltpu.VMEM((BLK, F), jnp.float32),
            pltpu.VMEM_SHARED((N_PAD, F), jnp.float32),
        ],
        compiler_params=_SC_PARAMS,
    )
    return k(vals16, idx2d, zeros_hbm)


BLK2 = 2000
NBLK2 = EPW // BLK2


def _repack_ea_body(ea_t, zro, out, plane_v, rows_v):
    c = lax.axis_index("c")
    s = lax.axis_index("s")
    wid = s * NC + c
    pltpu.sync_copy(zro.at[pl.ds(0, BLK2)], rows_v)
    iota = lax.iota(jnp.int32, 16)

    def body(b, carry):
        e0 = (wid * NBLK2 + b) * BLK2
        pltpu.sync_copy(ea_t.at[:, pl.ds(e0, BLK2)], plane_v)

        def grp(g, carry2):
            base = g * 16
            ridx = iota + base
            for a in range(6):
                vals = plane_v[a, pl.ds(base, 16)]
                plsc.store_scatter(rows_v, [ridx, jnp.full((16,), a, jnp.int32)],
                                   vals)
            return carry2

        lax.fori_loop(0, BLK2 // 16, grp, 0)
        pltpu.sync_copy(rows_v, out.at[pl.ds(e0, BLK2)])
        return carry

    lax.fori_loop(0, NBLK2, body, 0)


def _sc_repack_ea(ea_t, zeros_hbm):
    k = pl.kernel(
        _repack_ea_body,
        out_type=jax.ShapeDtypeStruct((E, F), jnp.float32),
        mesh=_sc_mesh(),
        scratch_types=[
            pltpu.VMEM((6, BLK2), jnp.float32),
            pltpu.VMEM((BLK2, F), jnp.float32),
        ],
        compiler_params=_SC_PARAMS_NL,
    )
    return k(ea_t, zeros_hbm)


def _unpack_er_body(er16, out, rows_v, plane_v):
    c = lax.axis_index("c")
    s = lax.axis_index("s")
    wid = s * NC + c
    iota = lax.iota(jnp.int32, 16)

    def body(b, carry):
        e0 = (wid * NBLK2 + b) * BLK2
        pltpu.sync_copy(er16.at[pl.ds(e0, BLK2)], rows_v)

        def grp(g, carry2):
            base = g * 16
            ridx = iota + base
            for f in range(3):
                vals = plsc.load_gather(
                    rows_v, [ridx, jnp.full((16,), f, jnp.int32)])
                plane_v[f, pl.ds(base, 16)] = vals
            return carry2

        lax.fori_loop(0, BLK2 // 16, grp, 0)
        pltpu.sync_copy(plane_v, out.at[:, pl.ds(e0, BLK2)])
        return carry

    lax.fori_loop(0, NBLK2, body, 0)


def _sc_unpack_er(er16):
    k = pl.kernel(
        _unpack_er_body,
        out_type=jax.ShapeDtypeStruct((3, E), jnp.float32),
        mesh=_sc_mesh(),
        scratch_types=[
            pltpu.VMEM((BLK2, F), jnp.float32),
            pltpu.VMEM((3, BLK2), jnp.float32),
        ],
        compiler_params=_SC_PARAMS_NL,
    )
    return k(er16)


def _embed_body(x_ref, w_ref, b_ref, o_ref):
    o_ref[...] = _MM(x_ref[...], w_ref[...]) + b_ref[...]


def _tc_embed(x_pad, wnp, bnp):
    return pl.pallas_call(
        _embed_body,
        grid=(N_PAD // NBR,),
        in_specs=[
            pl.BlockSpec((NBR, 12), lambda i: (i, 0)),
            pl.BlockSpec((12, F), lambda i: (0, 0)),
            pl.BlockSpec((1, F), lambda i: (0, 0)),
        ],
        out_specs=pl.BlockSpec((NBR, F), lambda i: (i, 0)),
        out_shape=jax.ShapeDtypeStruct((N_PAD, F), jnp.float32),
    )(x_pad, wnp, bnp)


def _edge1_body(ein, gs, gd, we, wa, wb, wc, w2, be, b1, b2, er_ref):
    xe = _MM(ein[...], we[...]) + be[...]
    h1 = jnp.maximum(
        _MM(gs[...], wa[...]) + _MM(xe, wb[...])
        + _MM(gd[...], wc[...]) + b1[...], 0.0)
    er_ref[...] = jnp.maximum(_MM(h1, w2[...]) + b2[...], 0.0)


def _edge2_body(ein, gs, gd, wa, wb, wc, w2, b1, b2, er_ref):
    h1 = jnp.maximum(
        _MM(gs[...], wa[...]) + _MM(ein[...], wb[...])
        + _MM(gd[...], wc[...]) + b1[...], 0.0)
    er_ref[...] = jnp.maximum(_MM(h1, w2[...]) + b2[...], 0.0)


def _edge3_body(ein, gs, gd, wa, wb, wc, w2, w3, b1, b2, b3, er_ref):
    h1 = jnp.maximum(
        _MM(gs[...], wa[...]) + _MM(ein[...], wb[...])
        + _MM(gd[...], wc[...]) + b1[...], 0.0)
    h2 = jnp.maximum(_MM(h1, w2[...]) + b2[...], 0.0)
    er_ref[...] = _MM(h2, w3[...]) + b3[...]


def _tc_edge(body, ein_p, gs_p, gd_p, mats, vecs):
    nmat, nvec = len(mats), len(vecs)
    big = pl.BlockSpec((EBR, 128), lambda i: (i, 0))
    wsp = pl.BlockSpec((128, 128), lambda i: (0, 0))
    bsp = pl.BlockSpec((1, 128), lambda i: (0, 0))
    return pl.pallas_call(
        body,
        grid=(EROWS // EBR,),
        in_specs=[big, big, big] + [wsp] * nmat + [bsp] * nvec,
        out_specs=big,
        out_shape=jax.ShapeDtypeStruct((EROWS, 128), jnp.float32),
    )(ein_p, gs_p, gd_p, *mats, *vecs)


def _node2_body(ga, ea_, na, nb, n2, c1, c2, o_ref):
    s1 = ga[0] + ga[1]
    s2 = ea_[0] + ea_[1]
    h = jnp.maximum(_MM(s1, na[...]) + _MM(s2, nb[...]) + c1[...], 0.0)
    o_ref[...] = jnp.maximum(_MM(h, n2[...]) + c2[...], 0.0)


def _node3_body(ga, ea_, na, nb, n2, n3, c1, c2, c3, o_ref):
    s1 = ga[0] + ga[1]
    s2 = ea_[0] + ea_[1]
    h1 = jnp.maximum(_MM(s1, na[...]) + _MM(s2, nb[...]) + c1[...], 0.0)
    h2 = jnp.maximum(_MM(h1, n2[...]) + c2[...], 0.0)
    o_ref[...] = _MM(h2, n3[...]) + c3[...]


def _tc_node(body, gsacc, eracc, mats, vecs):
    asp = pl.BlockSpec((NC, NBR, F), lambda i: (0, i, 0))
    wsp = pl.BlockSpec((F, F), lambda i: (0, 0))
    bsp = pl.BlockSpec((1, F), lambda i: (0, 0))
    return pl.pallas_call(
        body,
        grid=(N_PAD // NBR,),
        in_specs=[asp, asp] + [wsp] * len(mats) + [bsp] * len(vecs),
        out_specs=pl.BlockSpec((NBR, F), lambda i: (i, 0)),
        out_shape=jax.ShapeDtypeStruct((N_PAD, F), jnp.float32),
    )(gsacc, eracc, *mats, *vecs)


def _prep(params):
    f32 = jnp.float32
    Wn, bn = params['emb_n']
    We, be = params['emb_e']
    (W1, b1), (W2, b2) = params['c1_e']
    (N1, c1), (N2, c2) = params['c1_n']
    (V1, d1), (V2, d2) = params['c2_e']
    (M1, e1), (M2, e2) = params['c2_n']
    (U1, f1), (U2, f2), (U3, f3) = params['c3_e']
    (P1, g1), (P2, g2), (P3, g3) = params['c3_n']

    p = {}
    p['emb_w'] = _padc(Wn.astype(f32), F)
    p['emb_b'] = _padc(bn.astype(f32)[None, :], F)

    p['e1_we'] = _k8(_padr(_padc(We.astype(f32), F), F))
    p['e1_be'] = _b8(_padc(be.astype(f32)[None, :], F)[0])
    p['e1_wa'] = _k8(_padr(W1[0:12], F))
    p['e1_wb'] = _k8(_padr(W1[12:18], F))
    p['e1_wc'] = _k8(_padr(W1[18:30], F))
    p['e1_w2'] = _k8(W2)
    p['e1_b1'] = _b8(b1)
    p['e1_b2'] = _b8(b2)
    p['n1_a'] = _padr(N1[0:12].astype(f32), F)
    p['n1_b'] = N1[12:28].astype(f32)
    p['n1_w2'] = N2.astype(f32)
    p['n1_c1'] = c1.astype(f32)[None, :]
    p['n1_c2'] = c2.astype(f32)[None, :]

    p['e2_wa'] = _k8(V1[0:16])
    p['e2_wb'] = _k8(V1[16:32])
    p['e2_wc'] = _k8(V1[32:48])
    p['e2_w2'] = _k8(V2)
    p['e2_b1'] = _b8(d1)
    p['e2_b2'] = _b8(d2)
    p['n2_a'] = M1[0:16].astype(f32)
    p['n2_b'] = M1[16:32].astype(f32)
    p['n2_w2'] = M2.astype(f32)
    p['n2_c1'] = e1.astype(f32)[None, :]
    p['n2_c2'] = e2.astype(f32)[None, :]

    p['e3_wa'] = _k8(U1[0:16])
    p['e3_wb'] = _k8(U1[16:32])
    p['e3_wc'] = _k8(U1[32:48])
    p['e3_w2'] = _k8(U2)
    p['e3_w3'] = _k8(_padc(U3, F))
    p['e3_b1'] = _b8(f1)
    p['e3_b2'] = _b8(f2)
    p['e3_b3'] = _b8(_padc(f3[None, :], F)[0])
    p['n3_a'] = P1[0:16].astype(f32)
    p['n3_b'] = _padr(P1[16:19].astype(f32), F)
    p['n3_w2'] = P2.astype(f32)
    p['n3_w3'] = _padc(P3.astype(f32), F)
    p['n3_c1'] = g1.astype(f32)[None, :]
    p['n3_c2'] = g2.astype(f32)[None, :]
    p['n3_c3'] = _padc(g3.astype(f32)[None, :], F)
    return p


def _conv(tbl, ein_p, src2d, dst2d, zeros_hbm, p, tag):
    gs16 = _sc_gather(tbl, src2d)
    gs = gs16.reshape(EROWS, 128)
    gd = _sc_gather(tbl, dst2d).reshape(EROWS, 128)
    gsacc = _sc_scatter_add(gs16, dst2d, zeros_hbm)
    if tag == 'e1':
        er_p = _tc_edge(
            _edge1_body, ein_p, gs, gd,
            [p['e1_we'], p['e1_wa'], p['e1_wb'], p['e1_wc'], p['e1_w2']],
            [p['e1_be'], p['e1_b1'], p['e1_b2']])
    elif tag == 'e2':
        er_p = _tc_edge(
            _edge2_body, ein_p, gs, gd,
            [p['e2_wa'], p['e2_wb'], p['e2_wc'], p['e2_w2']],
            [p['e2_b1'], p['e2_b2']])
    else:
        er_p = _tc_edge(
            _edge3_body, ein_p, gs, gd,
            [p['e3_wa'], p['e3_wb'], p['e3_wc'], p['e3_w2'], p['e3_w3']],
            [p['e3_b1'], p['e3_b2'], p['e3_b3']])
    eracc = _sc_scatter_add(er_p.reshape(E, F), dst2d, zeros_hbm)
    return er_p, gsacc, eracc


def kernel(x, edge_attr, edge_index, params):
    p = _prep(params)
    src2d = edge_index[0].astype(jnp.int32).reshape(E // SUB, SUB)
    dst2d = edge_index[1].astype(jnp.int32).reshape(E // SUB, SUB)
    zeros_hbm = jnp.zeros((N_PAD, F), jnp.float32)

    ea_p = _sc_repack_ea(edge_attr.astype(jnp.float32).T,
                         zeros_hbm).reshape(EROWS, 128)
    x_pad = jnp.pad(x.astype(jnp.float32), ((0, N_PAD - N_NODES), (0, 0)))

    tbl1 = _tc_embed(x_pad, p['emb_w'], p['emb_b'])

    er1, ga1, ea1 = _conv(tbl1, ea_p, src2d, dst2d, zeros_hbm, p, 'e1')
    tbl2 = _tc_node(_node2_body, ga1, ea1, [p['n1_a'], p['n1_b'], p['n1_w2']],
                    [p['n1_c1'], p['n1_c2']])

    er2, ga2, ea2 = _conv(tbl2, er1, src2d, dst2d, zeros_hbm, p, 'e2')
    tbl3 = _tc_node(_node2_body, ga2, ea2, [p['n2_a'], p['n2_b'], p['n2_w2']],
                    [p['n2_c1'], p['n2_c2']])

    er3, ga3, ea3 = _conv(tbl3, er2, src2d, dst2d, zeros_hbm, p, 'e3')
    out_n = _tc_node(_node3_body, ga3, ea3,
                     [p['n3_a'], p['n3_b'], p['n3_w2'], p['n3_w3']],
                     [p['n3_c1'], p['n3_c2'], p['n3_c3']])

    xn_out = out_n[:N_NODES, :3]
    xe_out = _sc_unpack_er(er3.reshape(E, F)).T
    return (xn_out, xe_out)
